# Initial kernel scaffold; baseline (speedup 1.0000x reference)
import jax
import jax.numpy as jnp
from jax.experimental import pallas as pl


def kernel(x_bridge, x_road, edge_index_b2r, edge_index_r2b, W_enc_bridge, b_enc_bridge, W_enc_road, b_enc_road, Wl_b2r, bl_b2r, Wr_b2r, Wl_r2b, bl_r2b, Wr_r2b, W_out, b_out):
    raise NotImplementedError("write your pallas kernel here")


# speedup vs baseline: 10.5417x; 10.5417x over previous
import functools

import jax
import jax.numpy as jnp
from jax import lax
from jax.experimental import pallas as pl
from jax.experimental.pallas import tpu as pltpu
from jax.experimental.pallas import tpu_sc as plsc

N = 100000
E = 1600000
D_IN = 128
H = 32
HH = 16

NUM_CORES = 2
NUM_SUBCORES = 16
K = 512
C = 196
PER_SUB = K * C
E_PAD = PER_SUB * NUM_SUBCORES
ACC_N = 100096

ROW_BLK = 2000


def _enc_body(xb_ref, xr_ref, wb_ref, bb_ref, wr_ref, br_ref,
              hb_ref, lo_ref, hi_ref):
    hb = jnp.dot(xb_ref[...], wb_ref[...], preferred_element_type=jnp.float32)
    hb = jnp.maximum(hb + bb_ref[...], 0.0)
    hr = jnp.dot(xr_ref[...], wr_ref[...], preferred_element_type=jnp.float32)
    hr = jnp.maximum(hr + br_ref[...], 0.0)
    hb_ref[...] = hb
    lo_ref[...] = hr[:, :HH]
    hi_ref[...] = hr[:, HH:]


def _fin_body(lo_ref, hi_ref, cnt_ref, hb_ref, wllo_ref, wlhi_ref, bl_ref,
              wr_ref, wo_ref, bo_ref, out_ref):
    c = jnp.maximum(cnt_ref[...], 1.0)
    ml = lo_ref[...] / c
    mh = hi_ref[...] / c
    t = (jnp.dot(ml, wllo_ref[...], preferred_element_type=jnp.float32)
         + jnp.dot(mh, wlhi_ref[...], preferred_element_type=jnp.float32)
         + jnp.dot(hb_ref[...], wr_ref[...], preferred_element_type=jnp.float32)
         + bl_ref[...])
    u = jnp.maximum(t, 0.0)
    out_ref[...] = jnp.dot(u, wo_ref[...],
                           preferred_element_type=jnp.float32) + bo_ref[...]


def _sc_body(src_hbm, dst_hbm, hrlo_hbm, hrhi_hbm,
             slo_hbm, shi_hbm, cnt_hbm,
             acc, cacc, sidx, didx, rows, ones_v, zc, sem):
    c = lax.axis_index("c")
    s = lax.axis_index("s")

    def _rows_fill(i, carry):
        rows[i] = jnp.zeros((16,), jnp.float32)
        return carry
    lax.fori_loop(0, K, _rows_fill, 0)

    def _zc_fill(i, carry):
        zc[pl.ds(i * 16, 16)] = jnp.zeros((16,), jnp.float32)
        return carry
    lax.fori_loop(0, K // 16, _zc_fill, 0)

    def _ones_fill(i, carry):
        ones_v[pl.ds(i * 16, 16)] = jnp.ones((16,), jnp.float32)
        return carry
    lax.fori_loop(0, K // 16, _ones_fill, 0)

    tile_rows = ACC_N // NUM_SUBCORES
    for j in range(12):
        pltpu.sync_copy(rows, acc.at[pl.ds(s * tile_rows + j * K, K)])
        pltpu.sync_copy(zc, cacc.at[pl.ds(s * tile_rows + j * K, K)])
    last = tile_rows - K
    pltpu.sync_copy(rows, acc.at[pl.ds(s * tile_rows + last, K)])
    pltpu.sync_copy(zc, cacc.at[pl.ds(s * tile_rows + last, K)])

    plsc.subcore_barrier()

    base = s * PER_SUB

    def _chunk(j, carry):
        off = base + j * K
        pltpu.sync_copy(src_hbm.at[pl.ds(off, K)], sidx)
        pltpu.sync_copy(dst_hbm.at[pl.ds(off, K)], didx)

        @pl.when(c == 0)
        def _():
            pltpu.async_copy(hrlo_hbm.at[sidx], rows, sem).wait()
            pltpu.sync_copy(rows, acc.at[didx], add=True)

        @pl.when(c == 1)
        def _():
            pltpu.async_copy(hrhi_hbm.at[sidx], rows, sem).wait()
            pltpu.sync_copy(rows, acc.at[didx], add=True)
            pltpu.sync_copy(ones_v, cacc.at[didx], add=True)

        return carry
    lax.fori_loop(0, C, _chunk, 0)

    plsc.subcore_barrier()

    @pl.when(c == 0)
    def _():
        pltpu.sync_copy(acc.at[pl.ds(s * tile_rows, tile_rows)],
                        slo_hbm.at[pl.ds(s * tile_rows, tile_rows)])

    @pl.when(c == 1)
    def _():
        pltpu.sync_copy(acc.at[pl.ds(s * tile_rows, tile_rows)],
                        shi_hbm.at[pl.ds(s * tile_rows, tile_rows)])
        pltpu.sync_copy(cacc.at[pl.ds(s * tile_rows, tile_rows)],
                        cnt_hbm.at[pl.ds(s * tile_rows, tile_rows)])


def kernel(x_bridge, x_road, edge_index_b2r, edge_index_r2b,
           W_enc_bridge, b_enc_bridge, W_enc_road, b_enc_road,
           Wl_b2r, bl_b2r, Wr_b2r, Wl_r2b, bl_r2b, Wr_r2b,
           W_out, b_out):
    grid = N // ROW_BLK
    row_spec2 = pl.BlockSpec((ROW_BLK, D_IN), lambda i: (i, 0))
    w_spec = pl.BlockSpec((D_IN, H), lambda i: (0, 0))
    b_spec = pl.BlockSpec((1, H), lambda i: (0, 0))
    hb, hr_lo, hr_hi = pl.pallas_call(
        _enc_body,
        grid=(grid,),
        in_specs=[row_spec2, row_spec2, w_spec, b_spec, w_spec, b_spec],
        out_specs=[
            pl.BlockSpec((ROW_BLK, H), lambda i: (i, 0)),
            pl.BlockSpec((ROW_BLK, HH), lambda i: (i, 0)),
            pl.BlockSpec((ROW_BLK, HH), lambda i: (i, 0)),
        ],
        out_shape=[
            jax.ShapeDtypeStruct((N, H), jnp.float32),
            jax.ShapeDtypeStruct((N, HH), jnp.float32),
            jax.ShapeDtypeStruct((N, HH), jnp.float32),
        ],
    )(x_bridge, x_road,
      W_enc_bridge, b_enc_bridge.reshape(1, H),
      W_enc_road, b_enc_road.reshape(1, H))

    src = edge_index_r2b[0].astype(jnp.int32)
    dst = edge_index_r2b[1].astype(jnp.int32)
    pad = E_PAD - E
    src_p = jnp.concatenate([src, jnp.zeros((pad,), jnp.int32)])
    dst_p = jnp.concatenate([dst, jnp.full((pad,), N, jnp.int32)])

    mesh = plsc.VectorSubcoreMesh(core_axis_name="c", subcore_axis_name="s")
    sc_fn = pl.kernel(
        _sc_body,
        out_type=(
            jax.ShapeDtypeStruct((ACC_N, HH), jnp.float32),
            jax.ShapeDtypeStruct((ACC_N, HH), jnp.float32),
            jax.ShapeDtypeStruct((ACC_N,), jnp.float32),
        ),
        mesh=mesh,
        scratch_types=[
            pltpu.VMEM_SHARED((ACC_N, HH), jnp.float32),
            pltpu.VMEM_SHARED((ACC_N,), jnp.float32),
            pltpu.VMEM((K,), jnp.int32),
            pltpu.VMEM((K,), jnp.int32),
            pltpu.VMEM((K, HH), jnp.float32),
            pltpu.VMEM((K,), jnp.float32),
            pltpu.VMEM((K,), jnp.float32),
            pltpu.SemaphoreType.DMA,
        ],
        compiler_params=pltpu.CompilerParams(use_tc_tiling_on_sc=False),
    )
    sum_lo, sum_hi, cnt = sc_fn(src_p, dst_p, hr_lo, hr_hi)

    out = pl.pallas_call(
        _fin_body,
        grid=(grid,),
        in_specs=[
            pl.BlockSpec((ROW_BLK, HH), lambda i: (i, 0)),
            pl.BlockSpec((ROW_BLK, HH), lambda i: (i, 0)),
            pl.BlockSpec((ROW_BLK, 1), lambda i: (i, 0)),
            pl.BlockSpec((ROW_BLK, H), lambda i: (i, 0)),
            pl.BlockSpec((HH, H), lambda i: (0, 0)),
            pl.BlockSpec((HH, H), lambda i: (0, 0)),
            pl.BlockSpec((1, H), lambda i: (0, 0)),
            pl.BlockSpec((H, H), lambda i: (0, 0)),
            pl.BlockSpec((H, 1), lambda i: (0, 0)),
            pl.BlockSpec((1, 1), lambda i: (0, 0)),
        ],
        out_specs=pl.BlockSpec((ROW_BLK, 1), lambda i: (i, 0)),
        out_shape=jax.ShapeDtypeStruct((N, 1), jnp.float32),
    )(sum_lo, sum_hi, cnt[:N].reshape(N, 1), hb,
      Wl_r2b[:HH], Wl_r2b[HH:], bl_r2b.reshape(1, H),
      Wr_r2b, W_out, b_out.reshape(1, 1))
    return out
